# speedup vs baseline: 1.5049x; 1.5049x over previous
import functools

import jax
import jax.numpy as jnp
from jax import lax
from jax.experimental import pallas as pl
from jax.experimental.pallas import tpu as pltpu
from jax.experimental.pallas import tpu_sc as plsc

_D = 1024
_NC = 2
_NS = 16
_NW = _NC * _NS


@functools.lru_cache(maxsize=None)
def _build_gather(n, chunk):
    n_per_w = n // _NW
    nchunk = n_per_w // chunk
    mesh = plsc.VectorSubcoreMesh(core_axis_name="c", subcore_axis_name="s")

    @functools.partial(
        pl.kernel,
        mesh=mesh,
        out_type=jax.ShapeDtypeStruct((n, _D), jnp.float32),
        scratch_types=[
            pltpu.VMEM((nchunk, chunk), jnp.int32),
            pltpu.VMEM((chunk, _D), jnp.float32),
            pltpu.SemaphoreType.DMA,
        ],
    )
    def gather_kernel(idx_hbm, table_hbm, out_hbm, idx_v, buf, sem):
        wid = lax.axis_index("s") * _NC + lax.axis_index("c")
        base = wid * n_per_w
        pltpu.sync_copy(idx_hbm.at[wid], idx_v)
        for c in range(nchunk):
            pltpu.async_copy(table_hbm.at[idx_v.at[c]], buf, sem).wait()
            pltpu.sync_copy(buf, out_hbm.at[pl.ds(base + c * chunk, chunk)])

    return gather_kernel


def kernel(x, table):
    b, s = x.shape
    n = b * s
    chunk = 64
    idx = x.reshape(_NW, (n // _NW) // chunk, chunk)
    out = _build_gather(n, chunk)(idx, table)
    return out.reshape(b, s, _D)


# speedup vs baseline: 1.5308x; 1.0172x over previous
import functools

import jax
import jax.numpy as jnp
from jax import lax
from jax.experimental import pallas as pl
from jax.experimental.pallas import tpu as pltpu
from jax.experimental.pallas import tpu_sc as plsc

_D = 1024
_NC = 2
_NS = 16
_NW = _NC * _NS


@functools.lru_cache(maxsize=None)
def _build_gather(n, chunk):
    n_per_w = n // _NW
    nchunk = n_per_w // chunk
    mesh = plsc.VectorSubcoreMesh(core_axis_name="c", subcore_axis_name="s")

    @functools.partial(
        pl.kernel,
        mesh=mesh,
        out_type=jax.ShapeDtypeStruct((n, _D), jnp.float32),
        scratch_types=[
            pltpu.VMEM((nchunk, chunk), jnp.int32),
            pltpu.VMEM((chunk, _D), jnp.float32),
            pltpu.VMEM((chunk, _D), jnp.float32),
            pltpu.SemaphoreType.DMA,
            pltpu.SemaphoreType.DMA,
            pltpu.SemaphoreType.DMA,
            pltpu.SemaphoreType.DMA,
        ],
    )
    def gather_kernel(idx_hbm, table_hbm, out_hbm, idx_v, buf0, buf1,
                      g0, g1, s0, s1):
        wid = lax.axis_index("s") * _NC + lax.axis_index("c")
        base = wid * n_per_w
        bufs, gsem, ssem = (buf0, buf1), (g0, g1), (s0, s1)
        pltpu.sync_copy(idx_hbm.at[wid], idx_v)

        def start_gather(c):
            b = c % 2
            return pltpu.async_copy(table_hbm.at[idx_v.at[c]], bufs[b],
                                    gsem[b])

        def start_scatter(c):
            b = c % 2
            return pltpu.async_copy(
                bufs[b], out_hbm.at[pl.ds(base + c * chunk, chunk)], ssem[b])

        gd = [None] * nchunk
        sd = [None] * nchunk
        gd[0] = start_gather(0)
        for c in range(nchunk):
            if c + 1 < nchunk:
                if c - 1 >= 0:
                    sd[c - 1].wait()
                gd[c + 1] = start_gather(c + 1)
            gd[c].wait()
            sd[c] = start_scatter(c)
        if nchunk >= 2:
            sd[nchunk - 2].wait()
        sd[nchunk - 1].wait()

    return gather_kernel


def kernel(x, table):
    b, s = x.shape
    n = b * s
    chunk = 32
    idx = x.reshape(_NW, (n // _NW) // chunk, chunk)
    out = _build_gather(n, chunk)(idx, table)
    return out.reshape(b, s, _D)


# speedup vs baseline: 1.5369x; 1.0040x over previous
import functools

import jax
import jax.numpy as jnp
from jax import lax
from jax.experimental import pallas as pl
from jax.experimental.pallas import tpu as pltpu
from jax.experimental.pallas import tpu_sc as plsc

_D = 1024
_NC = 2
_NS = 16
_NW = _NC * _NS


@functools.lru_cache(maxsize=None)
def _build_gather(b, s, chunk):
    n = b * s
    n_per_w = n // _NW
    w_per_row = s // n_per_w
    nchunk = n_per_w // chunk
    mesh = plsc.VectorSubcoreMesh(core_axis_name="c", subcore_axis_name="s")

    @functools.partial(
        pl.kernel,
        mesh=mesh,
        out_type=jax.ShapeDtypeStruct((n, _D), jnp.float32),
        scratch_types=[
            pltpu.VMEM((n_per_w,), jnp.int32),
            pltpu.VMEM((chunk, _D), jnp.float32),
            pltpu.VMEM((chunk, _D), jnp.float32),
            pltpu.SemaphoreType.DMA,
            pltpu.SemaphoreType.DMA,
            pltpu.SemaphoreType.DMA,
            pltpu.SemaphoreType.DMA,
        ],
    )
    def gather_kernel(idx_hbm, table_hbm, out_hbm, idx_v, buf0, buf1,
                      g0, g1, s0, s1):
        wid = lax.axis_index("s") * _NC + lax.axis_index("c")
        base = wid * n_per_w
        row = wid // w_per_row
        col = (wid % w_per_row) * n_per_w
        bufs, gsem, ssem = (buf0, buf1), (g0, g1), (s0, s1)
        pltpu.sync_copy(idx_hbm.at[row, pl.ds(col, n_per_w)], idx_v)

        def start_gather(c):
            bb = c % 2
            return pltpu.async_copy(
                table_hbm.at[idx_v.at[pl.ds(c * chunk, chunk)]], bufs[bb],
                gsem[bb])

        def start_scatter(c):
            bb = c % 2
            return pltpu.async_copy(
                bufs[bb], out_hbm.at[pl.ds(base + c * chunk, chunk)],
                ssem[bb])

        gd = [None] * nchunk
        sd = [None] * nchunk
        gd[0] = start_gather(0)
        for c in range(nchunk):
            if c + 1 < nchunk:
                if c - 1 >= 0:
                    sd[c - 1].wait()
                gd[c + 1] = start_gather(c + 1)
            gd[c].wait()
            sd[c] = start_scatter(c)
        if nchunk >= 2:
            sd[nchunk - 2].wait()
        sd[nchunk - 1].wait()

    return gather_kernel


def kernel(x, table):
    b, s = x.shape
    out = _build_gather(b, s, 32)(x, table)
    return out.reshape(b, s, _D)


# speedup vs baseline: 1.5541x; 1.0112x over previous
import functools

import jax
import jax.numpy as jnp
from jax import lax
from jax.experimental import pallas as pl
from jax.experimental.pallas import tpu as pltpu
from jax.experimental.pallas import tpu_sc as plsc

_D = 1024
_NC = 2
_NS = 16
_NW = _NC * _NS


@functools.lru_cache(maxsize=None)
def _build_gather(b, s, chunk, nbuf):
    n = b * s
    n_per_w = n // _NW
    w_per_row = s // n_per_w
    nchunk = n_per_w // chunk
    mesh = plsc.VectorSubcoreMesh(core_axis_name="c", subcore_axis_name="s")

    @functools.partial(
        pl.kernel,
        mesh=mesh,
        out_type=jax.ShapeDtypeStruct((n, _D), jnp.float32),
        scratch_types=(
            [pltpu.VMEM((n_per_w,), jnp.int32)]
            + [pltpu.VMEM((chunk, _D), jnp.float32) for _ in range(nbuf)]
            + [pltpu.SemaphoreType.DMA for _ in range(2 * nbuf)]
        ),
    )
    def gather_kernel(idx_hbm, table_hbm, out_hbm, idx_v, *rest):
        bufs = rest[:nbuf]
        gsem = rest[nbuf:2 * nbuf]
        ssem = rest[2 * nbuf:3 * nbuf]
        wid = lax.axis_index("s") * _NC + lax.axis_index("c")
        base = wid * n_per_w
        row = wid // w_per_row
        col = (wid % w_per_row) * n_per_w
        pltpu.sync_copy(idx_hbm.at[row, pl.ds(col, n_per_w)], idx_v)

        def start_gather(c):
            bb = c % nbuf
            return pltpu.async_copy(
                table_hbm.at[idx_v.at[pl.ds(c * chunk, chunk)]], bufs[bb],
                gsem[bb])

        def start_scatter(c):
            bb = c % nbuf
            return pltpu.async_copy(
                bufs[bb], out_hbm.at[pl.ds(base + c * chunk, chunk)],
                ssem[bb])

        gd = [None] * nchunk
        sd = [None] * nchunk
        for c in range(min(nbuf, nchunk)):
            gd[c] = start_gather(c)
        for c in range(nchunk):
            gd[c].wait()
            sd[c] = start_scatter(c)
            nxt = c + nbuf
            if nxt < nchunk:
                sd[nxt - nbuf].wait()
                gd[nxt] = start_gather(nxt)
        for c in range(max(0, nchunk - nbuf), nchunk):
            sd[c].wait()

    return gather_kernel


def kernel(x, table):
    b, s = x.shape
    out = _build_gather(b, s, 16, 4)(x, table)
    return out.reshape(b, s, _D)


# speedup vs baseline: 1.5893x; 1.0227x over previous
import functools

import jax
import jax.numpy as jnp
from jax import lax
from jax.experimental import pallas as pl
from jax.experimental.pallas import tpu as pltpu
from jax.experimental.pallas import tpu_sc as plsc

_D = 1024
_NC = 2
_NS = 16
_NW = _NC * _NS


@functools.lru_cache(maxsize=None)
def _build_gather(b, s, chunk, nbuf):
    n = b * s
    n_per_w = n // _NW
    w_per_row = s // n_per_w
    nchunk = n_per_w // chunk
    mesh = plsc.VectorSubcoreMesh(core_axis_name="c", subcore_axis_name="s")

    @functools.partial(
        pl.kernel,
        mesh=mesh,
        out_type=jax.ShapeDtypeStruct((n, _D), jnp.float32),
        scratch_types=(
            [pltpu.VMEM((n_per_w,), jnp.int32)]
            + [pltpu.VMEM((chunk, _D), jnp.float32) for _ in range(nbuf)]
            + [pltpu.SemaphoreType.DMA for _ in range(2 * nbuf)]
        ),
    )
    def gather_kernel(idx_hbm, table_hbm, out_hbm, idx_v, *rest):
        bufs = rest[:nbuf]
        gsem = rest[nbuf:2 * nbuf]
        ssem = rest[2 * nbuf:3 * nbuf]
        wid = lax.axis_index("s") * _NC + lax.axis_index("c")
        base = wid * n_per_w
        row = wid // w_per_row
        col = (wid % w_per_row) * n_per_w
        pltpu.sync_copy(idx_hbm.at[row, pl.ds(col, n_per_w)], idx_v)

        def start_gather(c):
            bb = c % nbuf
            return pltpu.async_copy(
                table_hbm.at[idx_v.at[pl.ds(c * chunk, chunk)]], bufs[bb],
                gsem[bb])

        def start_scatter(c):
            bb = c % nbuf
            return pltpu.async_copy(
                bufs[bb], out_hbm.at[pl.ds(base + c * chunk, chunk)],
                ssem[bb])

        gd = [None] * nchunk
        sd = [None] * nchunk
        for c in range(min(nbuf, nchunk)):
            gd[c] = start_gather(c)
        for c in range(nchunk):
            gd[c].wait()
            sd[c] = start_scatter(c)
            nxt = c + nbuf
            if nxt < nchunk:
                sd[c].wait()
                gd[nxt] = start_gather(nxt)
        for c in range(max(0, nchunk - nbuf), nchunk):
            sd[c].wait()

    return gather_kernel


def kernel(x, table):
    b, s = x.shape
    out = _build_gather(b, s, 16, 7)(x, table)
    return out.reshape(b, s, _D)
